# baseline (speedup 1.0000x reference)
import functools

import jax
import jax.numpy as jnp
from jax import lax
from jax.experimental import pallas as pl
from jax.experimental.pallas import tpu as pltpu
from jax.experimental.pallas import tpu_sc as plsc

_N = 10000
_E = 320000
_D = 128
_T = 128
_L = 3

_NCORE = 2
_NSUB = 16
_CHW = 64
_NCH = 160
_NBUF = 4
_EPT = _CHW * _NCH
_EPAD = _EPT * _NCORE * _NSUB
_ACCR = 10112
_ZROWS = _ACCR // _NSUB


_HCH = _NCH // 4


def _segsum_body(h_hbm, src_hbm, dst_hbm, zeros_hbm, out_hbm,
                 src_v, dst_v, bufs, gsems, ssems, acc):
    c = lax.axis_index("c")
    s = lax.axis_index("s")

    def _gather(j, b):
        return pltpu.make_async_copy(h_hbm.at[src_v.at[j]], bufs[b],
                                     gsems[b])

    def _scatter_start(j, b):
        pltpu.async_copy(bufs[b], acc.at[dst_v.at[j]], ssems[b], add=True)

    def _scatter_wait(j, b):
        pltpu.make_async_copy(bufs[b], acc.at[dst_v.at[j]], ssems[b]).wait()

    pltpu.sync_copy(src_hbm.at[c, s, pl.ds(0, _HCH)], src_v)
    pltpu.sync_copy(dst_hbm.at[c, s, pl.ds(0, _HCH)], dst_v)
    for b in range(_NBUF):
        _gather(b, b).start()

    pltpu.sync_copy(zeros_hbm, acc.at[pl.ds(s * _ZROWS, _ZROWS)])
    plsc.subcore_barrier()

    for half in range(_NCH // _HCH):
        if half > 0:
            pltpu.sync_copy(src_hbm.at[c, s, pl.ds(half * _HCH, _HCH)],
                            src_v)
            pltpu.sync_copy(dst_hbm.at[c, s, pl.ds(half * _HCH, _HCH)],
                            dst_v)
            for b in range(_NBUF):
                _gather(b, b).start()

        @pl.loop(0, _HCH, step=_NBUF)
        def _(t):
            for b in range(_NBUF):
                j = t + b
                _gather(j, b).wait()
                _scatter_start(j, b)
                nb = (b + 1) % _NBUF

                @pl.when(jnp.logical_and(j + 1 - _NBUF >= 0,
                                         j + 1 < _HCH))
                def _():
                    _scatter_wait(j + 1 - _NBUF, nb)
                    _gather(j + 1, nb).start()

        for b in range(_NBUF):
            _scatter_wait(_HCH - _NBUF + b, b)

    plsc.subcore_barrier()
    pltpu.sync_copy(acc.at[pl.ds(s * _ZROWS, _ZROWS)],
                    out_hbm.at[c].at[pl.ds(s * _ZROWS, _ZROWS)])


@jax.jit
def _segment_partials(h, src, dst, zeros):
    mesh = plsc.VectorSubcoreMesh(core_axis_name="c", subcore_axis_name="s")
    k = pl.kernel(
        _segsum_body,
        out_type=jax.ShapeDtypeStruct((_NCORE, _ACCR, _D), jnp.float32),
        mesh=mesh,
        scratch_types=[
            pltpu.VMEM((_HCH, _CHW), jnp.int32),
            pltpu.VMEM((_HCH, _CHW), jnp.int32),
            [pltpu.VMEM((_CHW, _D), jnp.float32) for _ in range(_NBUF)],
            [pltpu.SemaphoreType.DMA for _ in range(_NBUF)],
            [pltpu.SemaphoreType.DMA for _ in range(_NBUF)],
            pltpu.VMEM_SHARED((_ACCR, _D), jnp.float32),
        ],
    )
    return k(h, src, dst, zeros)


def _dot(a, b):
    return jnp.dot(a, b, preferred_element_type=jnp.float32)


def _bn_mlp(hin, waT, ba, g1, b1, wbT, bb, g2, b2):
    y = _dot(hin, waT) + ba
    m = jnp.mean(y, axis=0, keepdims=True)
    v = jnp.mean((y - m) ** 2, axis=0, keepdims=True)
    y = jnp.maximum(g1 * (y - m) / jnp.sqrt(v + 1e-5) + b1, 0.0)
    y = _dot(y, wbT) + bb
    m = jnp.mean(y, axis=0, keepdims=True)
    v = jnp.mean((y - m) ** 2, axis=0, keepdims=True)
    return jnp.maximum(g2 * (y - m) / jnp.sqrt(v + 1e-5) + b2, 0.0)


def _stage0_body(x, waT, ba, g1, b1, wbT, bb, g2, b2, linT, linb,
                 h_out, out):
    h = _bn_mlp(x[...], waT[...], ba[...], g1[...], b1[...],
                wbT[...], bb[...], g2[...], b2[...])
    h_out[...] = h
    out[...] = _dot(h, linT[...]) + linb[...]


def _conv_body(scale, h, agg, out_in, waT, ba, g1, b1, wbT, bb, g2, b2,
               linT, linb, h_out, out):
    hin = h[...] * scale[...] + agg[0, :_N, :] + agg[1, :_N, :]
    hh = _bn_mlp(hin, waT[...], ba[...], g1[...], b1[...],
                 wbT[...], bb[...], g2[...], b2[...])
    h_out[...] = hh
    out[...] = out_in[...] + _dot(hh, linT[...]) + linb[...]


_f32 = jnp.float32
_stage0 = pl.pallas_call(
    _stage0_body,
    out_shape=(jax.ShapeDtypeStruct((_N, _D), _f32),
               jax.ShapeDtypeStruct((_N, _T), _f32)),
)
_conv = pl.pallas_call(
    _conv_body,
    out_shape=(jax.ShapeDtypeStruct((_N, _D), _f32),
               jax.ShapeDtypeStruct((_N, _T), _f32)),
)


def kernel(x, edge_index, fh_Wa, fh_ba, fh_g1, fh_b1, fh_Wb, fh_bb, fh_g2,
           fh_b2, conv_Wa, conv_ba, conv_g1, conv_b1, conv_Wb, conv_bb,
           conv_g2, conv_b2, eps, lin_W, lin_b):
    r = lambda a: a.reshape(1, -1)

    pad = _EPAD - _E
    src = jnp.concatenate([edge_index[0], jnp.zeros((pad,), jnp.int32)])
    dst = jnp.concatenate([edge_index[1], jnp.full((pad,), _N, jnp.int32)])
    src = src.reshape(_NCORE, _NSUB, _NCH, _CHW)
    dst = dst.reshape(_NCORE, _NSUB, _NCH, _CHW)
    zeros = jnp.zeros((_ZROWS, _D), _f32)

    h, out = _stage0(x, fh_Wa.T, r(fh_ba), r(fh_g1), r(fh_b1),
                     fh_Wb.T, r(fh_bb), r(fh_g2), r(fh_b2),
                     lin_W[0].T, r(lin_b[0]))
    for l in range(_L):
        agg = _segment_partials(h, src, dst, zeros)
        scale = (1.0 + eps[l]) * jnp.ones((1, _D), _f32)
        h, out = _conv(scale, h, agg, out,
                       conv_Wa[l].T, r(conv_ba[l]), r(conv_g1[l]),
                       r(conv_b1[l]), conv_Wb[l].T, r(conv_bb[l]),
                       r(conv_g2[l]), r(conv_b2[l]),
                       lin_W[l + 1].T, r(lin_b[l + 1]))
    return out


# speedup vs baseline: 1.0864x; 1.0864x over previous
import functools

import jax
import jax.numpy as jnp
from jax import lax
from jax.experimental import pallas as pl
from jax.experimental.pallas import tpu as pltpu
from jax.experimental.pallas import tpu_sc as plsc

_N = 10000
_E = 320000
_D = 128
_T = 128
_L = 3

_NCORE = 2
_NSUB = 16
_CHW = 128
_NCH = 80
_NBUF = 2
_EPT = _CHW * _NCH
_EPAD = _EPT * _NCORE * _NSUB
_ACCR = 10112
_ZROWS = _ACCR // _NSUB


_HCH = _NCH // 2


def _segsum_body(h_hbm, src_hbm, dst_hbm, zeros_hbm, out_hbm,
                 src_v, dst_v, bufs, gsems, acc):
    c = lax.axis_index("c")
    s = lax.axis_index("s")

    def _gather(j, b):
        return pltpu.make_async_copy(h_hbm.at[src_v.at[j]], bufs[b],
                                     gsems[b])

    pltpu.sync_copy(src_hbm.at[c, s, pl.ds(0, _HCH)], src_v)
    pltpu.sync_copy(dst_hbm.at[c, s, pl.ds(0, _HCH)], dst_v)
    _gather(0, 0).start()
    _gather(1, 1).start()

    pltpu.sync_copy(zeros_hbm, acc.at[pl.ds(s * _ZROWS, _ZROWS)])
    plsc.subcore_barrier()

    for blk in range(_NCH // _HCH):
        if blk > 0:
            pltpu.sync_copy(src_hbm.at[c, s, pl.ds(blk * _HCH, _HCH)],
                            src_v)
            pltpu.sync_copy(dst_hbm.at[c, s, pl.ds(blk * _HCH, _HCH)],
                            dst_v)
            _gather(0, 0).start()
            _gather(1, 1).start()

        @pl.loop(0, _HCH, step=2)
        def _(j):
            _gather(j, 0).wait()
            pltpu.sync_copy(bufs[0], acc.at[dst_v.at[j]], add=True)

            @pl.when(j + 2 < _HCH)
            def _():
                _gather(j + 2, 0).start()

            _gather(j + 1, 1).wait()
            pltpu.sync_copy(bufs[1], acc.at[dst_v.at[j + 1]], add=True)

            @pl.when(j + 3 < _HCH)
            def _():
                _gather(j + 3, 1).start()

    plsc.subcore_barrier()
    pltpu.sync_copy(acc.at[pl.ds(s * _ZROWS, _ZROWS)],
                    out_hbm.at[c].at[pl.ds(s * _ZROWS, _ZROWS)])


@jax.jit
def _segment_partials(h, src, dst, zeros):
    mesh = plsc.VectorSubcoreMesh(core_axis_name="c", subcore_axis_name="s")
    k = pl.kernel(
        _segsum_body,
        out_type=jax.ShapeDtypeStruct((_NCORE, _ACCR, _D), jnp.float32),
        mesh=mesh,
        scratch_types=[
            pltpu.VMEM((_HCH, _CHW), jnp.int32),
            pltpu.VMEM((_HCH, _CHW), jnp.int32),
            [pltpu.VMEM((_CHW, _D), jnp.float32) for _ in range(_NBUF)],
            [pltpu.SemaphoreType.DMA for _ in range(_NBUF)],
            pltpu.VMEM_SHARED((_ACCR, _D), jnp.float32),
        ],
    )
    return k(h, src, dst, zeros)


def _dot(a, b):
    return jnp.dot(a, b, preferred_element_type=jnp.float32)


def _bn_mlp(hin, waT, ba, g1, b1, wbT, bb, g2, b2):
    y = _dot(hin, waT) + ba
    m = jnp.mean(y, axis=0, keepdims=True)
    v = jnp.mean((y - m) ** 2, axis=0, keepdims=True)
    y = jnp.maximum(g1 * (y - m) / jnp.sqrt(v + 1e-5) + b1, 0.0)
    y = _dot(y, wbT) + bb
    m = jnp.mean(y, axis=0, keepdims=True)
    v = jnp.mean((y - m) ** 2, axis=0, keepdims=True)
    return jnp.maximum(g2 * (y - m) / jnp.sqrt(v + 1e-5) + b2, 0.0)


def _mlp0_body(x, waT, ba, g1, b1, wbT, bb, g2, b2, h_out):
    h_out[...] = _bn_mlp(x[...], waT[...], ba[...], g1[...], b1[...],
                         wbT[...], bb[...], g2[...], b2[...])


def _mlpc_body(scale, h, agg, waT, ba, g1, b1, wbT, bb, g2, b2, h_out):
    hin = h[...] * scale[...] + agg[0, :_N, :] + agg[1, :_N, :]
    h_out[...] = _bn_mlp(hin, waT[...], ba[...], g1[...], b1[...],
                         wbT[...], bb[...], g2[...], b2[...])


def _proj0_body(h, linT, linb, out):
    out[...] = _dot(h[...], linT[...]) + linb[...]


def _proj_body(out_in, h, linT, linb, out):
    out[...] = out_in[...] + _dot(h[...], linT[...]) + linb[...]


_f32 = jnp.float32
_mlp0 = pl.pallas_call(
    _mlp0_body, out_shape=jax.ShapeDtypeStruct((_N, _D), _f32))
_mlpc = pl.pallas_call(
    _mlpc_body, out_shape=jax.ShapeDtypeStruct((_N, _D), _f32))
_proj0 = pl.pallas_call(
    _proj0_body, out_shape=jax.ShapeDtypeStruct((_N, _T), _f32))
_proj = pl.pallas_call(
    _proj_body, out_shape=jax.ShapeDtypeStruct((_N, _T), _f32))


def kernel(x, edge_index, fh_Wa, fh_ba, fh_g1, fh_b1, fh_Wb, fh_bb, fh_g2,
           fh_b2, conv_Wa, conv_ba, conv_g1, conv_b1, conv_Wb, conv_bb,
           conv_g2, conv_b2, eps, lin_W, lin_b):
    r = lambda a: a.reshape(1, -1)

    pad = _EPAD - _E
    src = jnp.concatenate([edge_index[0], jnp.zeros((pad,), jnp.int32)])
    dst = jnp.concatenate([edge_index[1], jnp.full((pad,), _N, jnp.int32)])
    src = src.reshape(_NCORE, _NSUB, _NCH, _CHW)
    dst = dst.reshape(_NCORE, _NSUB, _NCH, _CHW)
    zeros = jnp.zeros((_ZROWS, _D), _f32)

    h = _mlp0(x, fh_Wa.T, r(fh_ba), r(fh_g1), r(fh_b1),
              fh_Wb.T, r(fh_bb), r(fh_g2), r(fh_b2))
    out = _proj0(h, lin_W[0].T, r(lin_b[0]))
    for l in range(_L):
        agg = _segment_partials(h, src, dst, zeros)
        scale = (1.0 + eps[l]) * jnp.ones((1, _D), _f32)
        h = _mlpc(scale, h, agg,
                  conv_Wa[l].T, r(conv_ba[l]), r(conv_g1[l]),
                  r(conv_b1[l]), conv_Wb[l].T, r(conv_bb[l]),
                  r(conv_g2[l]), r(conv_b2[l]))
        out = _proj(out, h, lin_W[l + 1].T, r(lin_b[l + 1]))
    return out


# speedup vs baseline: 1.1780x; 1.0843x over previous
import functools

import jax
import jax.numpy as jnp
from jax import lax
from jax.experimental import pallas as pl
from jax.experimental.pallas import tpu as pltpu
from jax.experimental.pallas import tpu_sc as plsc

_N = 10000
_E = 320000
_D = 128
_T = 128
_L = 3

_NCORE = 2
_NSUB = 16
_CHW = 128
_NCH = 80
_NBUF = 2
_EPT = _CHW * _NCH
_EPAD = _EPT * _NCORE * _NSUB
_ACCR = 10112
_ZROWS = _ACCR // _NSUB


_HCH = _NCH // 2


def _segsum_body(h_hbm, src_hbm, dst_hbm, zeros_hbm, out_hbm,
                 src_v, dst_v, bufs, gsems, acc):
    c = lax.axis_index("c")
    s = lax.axis_index("s")

    def _gather(j, b):
        return pltpu.make_async_copy(h_hbm.at[src_v.at[j]], bufs[b],
                                     gsems[b])

    pltpu.sync_copy(src_hbm.at[c, s, pl.ds(0, _HCH)], src_v)
    pltpu.sync_copy(dst_hbm.at[c, s, pl.ds(0, _HCH)], dst_v)
    _gather(0, 0).start()
    _gather(1, 1).start()

    pltpu.sync_copy(zeros_hbm, acc.at[pl.ds(s * _ZROWS, _ZROWS)])
    plsc.subcore_barrier()

    for blk in range(_NCH // _HCH):
        if blk > 0:
            pltpu.sync_copy(src_hbm.at[c, s, pl.ds(blk * _HCH, _HCH)],
                            src_v)
            pltpu.sync_copy(dst_hbm.at[c, s, pl.ds(blk * _HCH, _HCH)],
                            dst_v)
            _gather(0, 0).start()
            _gather(1, 1).start()

        @pl.loop(0, _HCH, step=2)
        def _(j):
            _gather(j, 0).wait()
            pltpu.sync_copy(bufs[0], acc.at[dst_v.at[j]], add=True)

            @pl.when(j + 2 < _HCH)
            def _():
                _gather(j + 2, 0).start()

            _gather(j + 1, 1).wait()
            pltpu.sync_copy(bufs[1], acc.at[dst_v.at[j + 1]], add=True)

            @pl.when(j + 3 < _HCH)
            def _():
                _gather(j + 3, 1).start()

    plsc.subcore_barrier()
    pltpu.sync_copy(acc.at[pl.ds(s * _ZROWS, _ZROWS)],
                    out_hbm.at[c].at[pl.ds(s * _ZROWS, _ZROWS)])


@jax.jit
def _segment_partials(h, src, dst, zeros):
    mesh = plsc.VectorSubcoreMesh(core_axis_name="c", subcore_axis_name="s")
    k = pl.kernel(
        _segsum_body,
        out_type=jax.ShapeDtypeStruct((_NCORE, _ACCR, _D), jnp.float32),
        mesh=mesh,
        scratch_types=[
            pltpu.VMEM((_HCH, _CHW), jnp.int32),
            pltpu.VMEM((_HCH, _CHW), jnp.int32),
            [pltpu.VMEM((_CHW, _D), jnp.float32) for _ in range(_NBUF)],
            [pltpu.SemaphoreType.DMA for _ in range(_NBUF)],
            pltpu.VMEM_SHARED((_ACCR, _D), jnp.float32),
        ],
    )
    return k(h, src, dst, zeros)


def _dot(a, b):
    return jnp.dot(a, b, preferred_element_type=jnp.float32)


def _bn_mlp(hin, waT, ba, g1, b1, wbT, bb, g2, b2):
    y = _dot(hin, waT) + ba
    m = jnp.mean(y, axis=0, keepdims=True)
    v = jnp.mean((y - m) ** 2, axis=0, keepdims=True)
    y = jnp.maximum(g1 * (y - m) / jnp.sqrt(v + 1e-5) + b1, 0.0)
    y = _dot(y, wbT) + bb
    m = jnp.mean(y, axis=0, keepdims=True)
    v = jnp.mean((y - m) ** 2, axis=0, keepdims=True)
    return jnp.maximum(g2 * (y - m) / jnp.sqrt(v + 1e-5) + b2, 0.0)


def _stage0_body(x, waT, ba, g1, b1, wbT, bb, g2, b2, linT, linb,
                 h_out, out):
    h = _bn_mlp(x[...], waT[...], ba[...], g1[...], b1[...],
                wbT[...], bb[...], g2[...], b2[...])
    h_out[...] = h
    out[...] = _dot(h, linT[...]) + linb[...]


def _conv_body(scale, h, agg, out_in, waT, ba, g1, b1, wbT, bb, g2, b2,
               linT, linb, h_out, out):
    hin = h[...] * scale[...] + agg[0, :_N, :] + agg[1, :_N, :]
    hh = _bn_mlp(hin, waT[...], ba[...], g1[...], b1[...],
                 wbT[...], bb[...], g2[...], b2[...])
    h_out[...] = hh
    out[...] = out_in[...] + _dot(hh, linT[...]) + linb[...]


_f32 = jnp.float32
_stage0 = pl.pallas_call(
    _stage0_body,
    out_shape=(jax.ShapeDtypeStruct((_N, _D), _f32),
               jax.ShapeDtypeStruct((_N, _T), _f32)),
)
_conv = pl.pallas_call(
    _conv_body,
    out_shape=(jax.ShapeDtypeStruct((_N, _D), _f32),
               jax.ShapeDtypeStruct((_N, _T), _f32)),
)


def kernel(x, edge_index, fh_Wa, fh_ba, fh_g1, fh_b1, fh_Wb, fh_bb, fh_g2,
           fh_b2, conv_Wa, conv_ba, conv_g1, conv_b1, conv_Wb, conv_bb,
           conv_g2, conv_b2, eps, lin_W, lin_b):
    r = lambda a: a.reshape(1, -1)

    pad = _EPAD - _E
    src = jnp.concatenate([edge_index[0], jnp.zeros((pad,), jnp.int32)])
    dst = jnp.concatenate([edge_index[1], jnp.full((pad,), _N, jnp.int32)])
    src = src.reshape(_NCORE, _NSUB, _NCH, _CHW)
    dst = dst.reshape(_NCORE, _NSUB, _NCH, _CHW)
    zeros = jnp.zeros((_ZROWS, _D), _f32)

    h, out = _stage0(x, fh_Wa.T, r(fh_ba), r(fh_g1), r(fh_b1),
                     fh_Wb.T, r(fh_bb), r(fh_g2), r(fh_b2),
                     lin_W[0].T, r(lin_b[0]))
    for l in range(_L):
        agg = _segment_partials(h, src, dst, zeros)
        scale = (1.0 + eps[l]) * jnp.ones((1, _D), _f32)
        h, out = _conv(scale, h, agg, out,
                       conv_Wa[l].T, r(conv_ba[l]), r(conv_g1[l]),
                       r(conv_b1[l]), conv_Wb[l].T, r(conv_bb[l]),
                       r(conv_g2[l]), r(conv_b2[l]),
                       lin_W[l + 1].T, r(lin_b[l + 1]))
    return out


# speedup vs baseline: 1.2536x; 1.0642x over previous
import functools

import jax
import jax.numpy as jnp
from jax import lax
from jax.experimental import pallas as pl
from jax.experimental.pallas import tpu as pltpu
from jax.experimental.pallas import tpu_sc as plsc

_N = 10000
_E = 320000
_D = 128
_T = 128
_L = 3

_NCORE = 2
_NSUB = 16
_CHW = 128
_NCH = 80
_NBUF = 2
_EPT = _CHW * _NCH
_EPAD = _EPT * _NCORE * _NSUB
_ACCR = 10240
_ZROWS = _ACCR // _NSUB


_HCH = _NCH // 2


def _segsum_body(h_hbm, src_hbm, dst_hbm, zeros_hbm, out_hbm,
                 src_v, dst_v, bufs, gsems, acc):
    c = lax.axis_index("c")
    s = lax.axis_index("s")

    def _gather(j, b):
        return pltpu.make_async_copy(h_hbm.at[src_v.at[j]], bufs[b],
                                     gsems[b])

    pltpu.sync_copy(zeros_hbm, acc.at[pl.ds(s * _ZROWS, _ZROWS)])
    plsc.subcore_barrier()

    for blk in range(_NCH // _HCH):
        pltpu.sync_copy(src_hbm.at[c, s, pl.ds(blk * _HCH, _HCH)], src_v)
        pltpu.sync_copy(dst_hbm.at[c, s, pl.ds(blk * _HCH, _HCH)], dst_v)
        _gather(0, 0).start()

        @pl.loop(0, _HCH, step=2)
        def _(j):
            _gather(j + 1, 1).start()
            _gather(j, 0).wait()
            pltpu.sync_copy(bufs[0], acc.at[dst_v.at[j]], add=True)

            @pl.when(j + 2 < _HCH)
            def _():
                _gather(j + 2, 0).start()

            _gather(j + 1, 1).wait()
            pltpu.sync_copy(bufs[1], acc.at[dst_v.at[j + 1]], add=True)

    plsc.subcore_barrier()
    pltpu.sync_copy(acc.at[pl.ds(s * _ZROWS, _ZROWS)],
                    out_hbm.at[c].at[pl.ds(s * _ZROWS, _ZROWS)])


@jax.jit
def _segment_partials(h, src, dst, zeros):
    mesh = plsc.VectorSubcoreMesh(core_axis_name="c", subcore_axis_name="s")
    k = pl.kernel(
        _segsum_body,
        out_type=jax.ShapeDtypeStruct((_NCORE, _ACCR, _D), jnp.float32),
        mesh=mesh,
        scratch_types=[
            pltpu.VMEM((_HCH, _CHW), jnp.int32),
            pltpu.VMEM((_HCH, _CHW), jnp.int32),
            [pltpu.VMEM((_CHW, _D), jnp.float32) for _ in range(_NBUF)],
            [pltpu.SemaphoreType.DMA for _ in range(_NBUF)],
            pltpu.VMEM_SHARED((_ACCR, _D), jnp.float32),
        ],
    )
    return k(h, src, dst, zeros)


def _dot(a, b):
    return jnp.dot(a, b, preferred_element_type=jnp.float32)


def _bn_mlp(hin, waT, ba, g1, b1, wbT, bb, g2, b2):
    y = _dot(hin, waT) + ba
    m = jnp.mean(y, axis=0, keepdims=True)
    v = jnp.mean((y - m) ** 2, axis=0, keepdims=True)
    y = jnp.maximum(g1 * (y - m) / jnp.sqrt(v + 1e-5) + b1, 0.0)
    y = _dot(y, wbT) + bb
    m = jnp.mean(y, axis=0, keepdims=True)
    v = jnp.mean((y - m) ** 2, axis=0, keepdims=True)
    return jnp.maximum(g2 * (y - m) / jnp.sqrt(v + 1e-5) + b2, 0.0)


def _stage0_body(x, waT, ba, g1, b1, wbT, bb, g2, b2, linT, linb,
                 h_out, out):
    h = _bn_mlp(x[...], waT[...], ba[...], g1[...], b1[...],
                wbT[...], bb[...], g2[...], b2[...])
    h_out[...] = h
    out[...] = _dot(h, linT[...]) + linb[...]


def _conv_body(scale, h, agg, out_in, waT, ba, g1, b1, wbT, bb, g2, b2,
               linT, linb, h_out, out):
    hin = h[...] * scale[...] + agg[0, :_N, :] + agg[1, :_N, :]
    hh = _bn_mlp(hin, waT[...], ba[...], g1[...], b1[...],
                 wbT[...], bb[...], g2[...], b2[...])
    h_out[...] = hh
    out[...] = out_in[...] + _dot(hh, linT[...]) + linb[...]


_f32 = jnp.float32
_stage0 = pl.pallas_call(
    _stage0_body,
    out_shape=(jax.ShapeDtypeStruct((_N, _D), _f32),
               jax.ShapeDtypeStruct((_N, _T), _f32)),
)
_conv = pl.pallas_call(
    _conv_body,
    out_shape=(jax.ShapeDtypeStruct((_N, _D), _f32),
               jax.ShapeDtypeStruct((_N, _T), _f32)),
)


def kernel(x, edge_index, fh_Wa, fh_ba, fh_g1, fh_b1, fh_Wb, fh_bb, fh_g2,
           fh_b2, conv_Wa, conv_ba, conv_g1, conv_b1, conv_Wb, conv_bb,
           conv_g2, conv_b2, eps, lin_W, lin_b):
    r = lambda a: a.reshape(1, -1)

    pad = _EPAD - _E
    src = jnp.concatenate([edge_index[0], jnp.zeros((pad,), jnp.int32)])
    dst = jnp.concatenate([edge_index[1], jnp.full((pad,), _N, jnp.int32)])
    src = src.reshape(_NCORE, _NSUB, _NCH, _CHW)
    dst = dst.reshape(_NCORE, _NSUB, _NCH, _CHW)
    zeros = jnp.zeros((_ZROWS, _D), _f32)

    h, out = _stage0(x, fh_Wa.T, r(fh_ba), r(fh_g1), r(fh_b1),
                     fh_Wb.T, r(fh_bb), r(fh_g2), r(fh_b2),
                     lin_W[0].T, r(lin_b[0]))
    for l in range(_L):
        agg = _segment_partials(h, src, dst, zeros)
        scale = (1.0 + eps[l]) * jnp.ones((1, _D), _f32)
        h, out = _conv(scale, h, agg, out,
                       conv_Wa[l].T, r(conv_ba[l]), r(conv_g1[l]),
                       r(conv_b1[l]), conv_Wb[l].T, r(conv_bb[l]),
                       r(conv_g2[l]), r(conv_b2[l]),
                       lin_W[l + 1].T, r(lin_b[l + 1]))
    return out
